# Initial kernel scaffold; baseline (speedup 1.0000x reference)
import functools

import jax
import jax.numpy as jnp
from jax import lax
from jax.experimental import pallas as pl
from jax.experimental.pallas import tpu as pltpu
from jax.experimental.pallas import tpu_sc as plsc

_N = 10000
_E = 320000
_D = 128
_C = 40
_CP = 64
_NP = 10240
_NC = 2
_NS = 16
_NW = _NC * _NS
_EPW = _E // _NW
_K = 80
_NCHUNK = _EPW // _K
_STRIP = _NP // _NS
_ZR = 128
_RB = 640

_MESH = dict(core_axis_name="c", subcore_axis_name="s",
             num_cores=_NC, num_subcores=_NS)


def _zero_fill_2d(ref, nrows, dim):
    zeros16 = jnp.zeros((16,), jnp.float32)
    per_row = dim // 16

    def body(i, carry):
        ref[i // per_row, pl.ds((i % per_row) * 16, 16)] = zeros16
        return carry

    lax.fori_loop(0, nrows * per_row, body, 0)


def _zero_fill_1d(ref, n):
    zeros16 = jnp.zeros((16,), jnp.float32)

    def body(i, carry):
        ref[pl.ds(i * 16, 16)] = zeros16
        return carry

    lax.fori_loop(0, n // 16, body, 0)


def _make_edge_scatter(dim):
    mesh = plsc.VectorSubcoreMesh(**_MESH)

    @functools.partial(
        pl.kernel,
        out_type=jax.ShapeDtypeStruct((_NC * _NP, dim), jnp.float32),
        mesh=mesh,
        scratch_types=[
            pltpu.VMEM((_K,), jnp.int32),
            pltpu.VMEM((_K,), jnp.int32),
            pltpu.VMEM((_K, dim), jnp.float32),
            pltpu.VMEM((_ZR, dim), jnp.float32),
            pltpu.VMEM_SHARED((_NP, dim), jnp.float32),
            pltpu.SemaphoreType.DMA,
        ],
    )
    def scatter_kernel(p_hbm, row_hbm, col_hbm, out_hbm,
                       row_v, col_v, rows_v, stage_v, acc, sem):
        c = lax.axis_index("c")
        s = lax.axis_index("s")
        wid = c * _NS + s
        strip = s * _STRIP

        _zero_fill_2d(stage_v, _ZR, dim)
        for z in range(_STRIP // _ZR):
            pltpu.sync_copy(stage_v, acc.at[pl.ds(strip + z * _ZR, _ZR), :])
        plsc.subcore_barrier()

        base = wid * _EPW

        def body(j, carry):
            off = base + j * _K
            pltpu.sync_copy(row_hbm.at[pl.ds(off, _K)], row_v)
            pltpu.sync_copy(col_hbm.at[pl.ds(off, _K)], col_v)
            pltpu.async_copy(p_hbm.at[row_v], rows_v, sem).wait()
            pltpu.sync_copy(rows_v, acc.at[col_v], add=True)
            return carry

        lax.fori_loop(0, _NCHUNK, body, 0)
        plsc.subcore_barrier()

        for z in range(_STRIP // _ZR):
            r0 = strip + z * _ZR
            pltpu.sync_copy(acc.at[pl.ds(r0, _ZR), :], stage_v)
            pltpu.sync_copy(stage_v, out_hbm.at[pl.ds(c * _NP + r0, _ZR), :])

    return scatter_kernel


def _make_degree():
    mesh = plsc.VectorSubcoreMesh(**_MESH)

    @functools.partial(
        pl.kernel,
        out_type=jax.ShapeDtypeStruct((_NC * _NP,), jnp.float32),
        mesh=mesh,
        scratch_types=[
            pltpu.VMEM((_K,), jnp.int32),
            pltpu.VMEM((_K,), jnp.float32),
            pltpu.VMEM((_STRIP,), jnp.float32),
            pltpu.VMEM_SHARED((_NP,), jnp.float32),
        ],
    )
    def degree_kernel(col_hbm, out_hbm, col_v, ones_v, stage_v, acc):
        c = lax.axis_index("c")
        s = lax.axis_index("s")
        wid = c * _NS + s
        strip = s * _STRIP

        ones16 = jnp.ones((16,), jnp.float32)
        for i in range(_K // 16):
            ones_v[pl.ds(i * 16, 16)] = ones16
        _zero_fill_1d(stage_v, _STRIP)
        pltpu.sync_copy(stage_v, acc.at[pl.ds(strip, _STRIP)])
        plsc.subcore_barrier()

        base = wid * _EPW

        def body(j, carry):
            off = base + j * _K
            pltpu.sync_copy(col_hbm.at[pl.ds(off, _K)], col_v)
            pltpu.sync_copy(ones_v, acc.at[col_v], add=True)
            return carry

        lax.fori_loop(0, _NCHUNK, body, 0)
        plsc.subcore_barrier()

        pltpu.sync_copy(acc.at[pl.ds(strip, _STRIP)], stage_v)
        pltpu.sync_copy(stage_v, out_hbm.at[pl.ds(c * _NP + strip, _STRIP)])

    return degree_kernel


_edge_scatter_d = _make_edge_scatter(_D)
_edge_scatter_c = _make_edge_scatter(_CP)
_degree = _make_degree()


def _p1_body(x_ref, w1_ref, deg_ref, p1_ref, dinv_ref):
    di = lax.rsqrt(deg_ref[...])
    p1_ref[...] = di * jnp.dot(x_ref[...], w1_ref[...],
                               preferred_element_type=jnp.float32)
    dinv_ref[...] = di


def _tc_p1(x_pad, w1, deg_col):
    grid = _NP // _RB
    return pl.pallas_call(
        _p1_body,
        grid=(grid,),
        in_specs=[
            pl.BlockSpec((_RB, _D), lambda i: (i, 0)),
            pl.BlockSpec((_D, _D), lambda i: (0, 0)),
            pl.BlockSpec((_RB, 1), lambda i: (i, 0)),
        ],
        out_specs=[
            pl.BlockSpec((_RB, _D), lambda i: (i, 0)),
            pl.BlockSpec((_RB, 1), lambda i: (i, 0)),
        ],
        out_shape=[
            jax.ShapeDtypeStruct((_NP, _D), jnp.float32),
            jax.ShapeDtypeStruct((_NP, 1), jnp.float32),
        ],
    )(x_pad, w1, deg_col)


def _p2_body(s1_ref, p1_ref, dinv_ref, b1_ref, w2_ref, p2_ref):
    di = dinv_ref[...]
    h = di * (s1_ref[0] + s1_ref[1] + p1_ref[...]) + b1_ref[...]
    h = jnp.maximum(h, 0.0)
    p2_ref[...] = di * jnp.dot(h, w2_ref[...],
                               preferred_element_type=jnp.float32)


def _tc_p2(s1, p1, dinv, b1_row, w2p):
    grid = _NP // _RB
    return pl.pallas_call(
        _p2_body,
        grid=(grid,),
        in_specs=[
            pl.BlockSpec((2, _RB, _D), lambda i: (0, i, 0)),
            pl.BlockSpec((_RB, _D), lambda i: (i, 0)),
            pl.BlockSpec((_RB, 1), lambda i: (i, 0)),
            pl.BlockSpec((1, _D), lambda i: (0, 0)),
            pl.BlockSpec((_D, _CP), lambda i: (0, 0)),
        ],
        out_specs=pl.BlockSpec((_RB, _CP), lambda i: (i, 0)),
        out_shape=jax.ShapeDtypeStruct((_NP, _CP), jnp.float32),
    )(s1, p1, dinv, b1_row, w2p)


def _final_body(s2_ref, p2_ref, dinv_ref, b2_ref, logp_ref, logits_ref):
    di = dinv_ref[...]
    lg = di * (s2_ref[0] + s2_ref[1] + p2_ref[...]) + b2_ref[...]
    icol = lax.broadcasted_iota(jnp.int32, (_RB, _CP), 1)
    neg = jnp.float32(-jnp.inf)
    lm = jnp.where(icol < _C, lg, neg)
    m = jnp.max(lm, axis=1, keepdims=True)
    e = jnp.where(icol < _C, jnp.exp(lm - m), 0.0)
    lse = m + jnp.log(jnp.sum(e, axis=1, keepdims=True))
    logp_ref[...] = lg - lse
    logits_ref[...] = lg


def _tc_final(s2, p2, dinv, b2_row):
    grid = _NP // _RB
    return pl.pallas_call(
        _final_body,
        grid=(grid,),
        in_specs=[
            pl.BlockSpec((2, _RB, _CP), lambda i: (0, i, 0)),
            pl.BlockSpec((_RB, _CP), lambda i: (i, 0)),
            pl.BlockSpec((_RB, 1), lambda i: (i, 0)),
            pl.BlockSpec((1, _CP), lambda i: (0, 0)),
        ],
        out_specs=[
            pl.BlockSpec((_RB, _CP), lambda i: (i, 0)),
            pl.BlockSpec((_RB, _CP), lambda i: (i, 0)),
        ],
        out_shape=[
            jax.ShapeDtypeStruct((_NP, _CP), jnp.float32),
            jax.ShapeDtypeStruct((_NP, _CP), jnp.float32),
        ],
    )(s2, p2, dinv, b2_row)


def kernel(x, edge_index, W1, b1, W2, b2):
    row = edge_index[0].astype(jnp.int32)
    col = edge_index[1].astype(jnp.int32)

    degp = _degree(col)
    deg_col = (degp[:_NP] + degp[_NP:] + 1.0).reshape(_NP, 1)

    x_pad = jnp.pad(x, ((0, _NP - _N), (0, 0)))
    p1, dinv = _tc_p1(x_pad, W1, deg_col)

    s1 = _edge_scatter_d(p1, row, col).reshape(_NC, _NP, _D)

    b1_row = b1.reshape(1, _D)
    w2p = jnp.pad(W2, ((0, 0), (0, _CP - _C)))
    p2 = _tc_p2(s1, p1, dinv, b1_row, w2p)

    s2 = _edge_scatter_c(p2, row, col).reshape(_NC, _NP, _CP)

    b2_row = jnp.pad(b2, (0, _CP - _C)).reshape(1, _CP)
    logp, logits = _tc_final(s2, p2, dinv, b2_row)

    return (logp[:_N, :_C], logits[:_N, :_C])


# speedup vs baseline: 13.4581x; 13.4581x over previous
import functools

import jax
import jax.numpy as jnp
from jax import lax
from jax.experimental import pallas as pl
from jax.experimental.pallas import tpu as pltpu
from jax.experimental.pallas import tpu_sc as plsc

_N = 10000
_E = 320000
_D = 128
_C = 40
_CP = 64
_NP = 10240
_NC = 2
_NS = 16
_NW = _NC * _NS
_EPW = _E // _NW
_K = 80
_NCHUNK = _EPW // _K
_STRIP = _NP // _NS
_ZR = 128
_RB = 640

_MESH = dict(core_axis_name="c", subcore_axis_name="s",
             num_cores=_NC, num_subcores=_NS)


def _zero_fill_2d(ref, nrows, dim):
    zeros16 = jnp.zeros((16,), jnp.float32)
    per_row = dim // 16

    def body(i, carry):
        ref[i // per_row, pl.ds((i % per_row) * 16, 16)] = zeros16
        return carry

    lax.fori_loop(0, nrows * per_row, body, 0)


def _zero_fill_1d(ref, n):
    zeros16 = jnp.zeros((16,), jnp.float32)

    def body(i, carry):
        ref[pl.ds(i * 16, 16)] = zeros16
        return carry

    lax.fori_loop(0, n // 16, body, 0)


def _make_edge_scatter(dim):
    mesh = plsc.VectorSubcoreMesh(**_MESH)

    @functools.partial(
        pl.kernel,
        out_type=jax.ShapeDtypeStruct((_NC * _NP, dim), jnp.float32),
        mesh=mesh,
        scratch_types=[
            pltpu.VMEM((_K,), jnp.int32),
            pltpu.VMEM((_K,), jnp.int32),
            pltpu.VMEM((_K, dim), jnp.float32),
            pltpu.VMEM((_ZR, dim), jnp.float32),
            pltpu.VMEM_SHARED((_NP, dim), jnp.float32),
            pltpu.SemaphoreType.DMA,
        ],
        compiler_params=pltpu.CompilerParams(use_tc_tiling_on_sc=False),
    )
    def scatter_kernel(p_hbm, row_hbm, col_hbm, out_hbm,
                       row_v, col_v, rows_v, stage_v, acc, sem):
        c = lax.axis_index("c")
        s = lax.axis_index("s")
        wid = c * _NS + s
        strip = s * _STRIP

        _zero_fill_2d(stage_v, _ZR, dim)
        for z in range(_STRIP // _ZR):
            pltpu.sync_copy(stage_v, acc.at[pl.ds(strip + z * _ZR, _ZR), :])
        plsc.subcore_barrier()

        base = wid * _EPW

        def body(j, carry):
            off = base + j * _K
            pltpu.sync_copy(row_hbm.at[pl.ds(off, _K)], row_v)
            pltpu.sync_copy(col_hbm.at[pl.ds(off, _K)], col_v)
            pltpu.async_copy(p_hbm.at[row_v], rows_v, sem).wait()
            pltpu.sync_copy(rows_v, acc.at[col_v], add=True)
            return carry

        lax.fori_loop(0, _NCHUNK, body, 0)
        plsc.subcore_barrier()

        for z in range(_STRIP // _ZR):
            r0 = strip + z * _ZR
            pltpu.sync_copy(acc.at[pl.ds(r0, _ZR), :], stage_v)
            pltpu.sync_copy(stage_v, out_hbm.at[pl.ds(c * _NP + r0, _ZR), :])

    return scatter_kernel


def _make_degree():
    mesh = plsc.VectorSubcoreMesh(**_MESH)

    @functools.partial(
        pl.kernel,
        out_type=jax.ShapeDtypeStruct((_NC * _NP,), jnp.float32),
        mesh=mesh,
        scratch_types=[
            pltpu.VMEM((_K,), jnp.int32),
            pltpu.VMEM((_K,), jnp.float32),
            pltpu.VMEM((_STRIP,), jnp.float32),
            pltpu.VMEM_SHARED((_NP,), jnp.float32),
        ],
    )
    def degree_kernel(col_hbm, out_hbm, col_v, ones_v, stage_v, acc):
        c = lax.axis_index("c")
        s = lax.axis_index("s")
        wid = c * _NS + s
        strip = s * _STRIP

        ones16 = jnp.ones((16,), jnp.float32)
        for i in range(_K // 16):
            ones_v[pl.ds(i * 16, 16)] = ones16
        _zero_fill_1d(stage_v, _STRIP)
        pltpu.sync_copy(stage_v, acc.at[pl.ds(strip, _STRIP)])
        plsc.subcore_barrier()

        base = wid * _EPW

        def body(j, carry):
            off = base + j * _K
            pltpu.sync_copy(col_hbm.at[pl.ds(off, _K)], col_v)
            pltpu.sync_copy(ones_v, acc.at[col_v], add=True)
            return carry

        lax.fori_loop(0, _NCHUNK, body, 0)
        plsc.subcore_barrier()

        pltpu.sync_copy(acc.at[pl.ds(strip, _STRIP)], stage_v)
        pltpu.sync_copy(stage_v, out_hbm.at[pl.ds(c * _NP + strip, _STRIP)])

    return degree_kernel


_edge_scatter_d = _make_edge_scatter(_D)
_edge_scatter_c = _make_edge_scatter(_CP)
_degree = _make_degree()


def _p1_body(x_ref, w1_ref, deg_ref, p1_ref, dinv_ref):
    di = lax.rsqrt(deg_ref[...])
    p1_ref[...] = di * jnp.dot(x_ref[...], w1_ref[...],
                               preferred_element_type=jnp.float32)
    dinv_ref[...] = di


def _tc_p1(x_pad, w1, deg_col):
    grid = _NP // _RB
    return pl.pallas_call(
        _p1_body,
        grid=(grid,),
        in_specs=[
            pl.BlockSpec((_RB, _D), lambda i: (i, 0)),
            pl.BlockSpec((_D, _D), lambda i: (0, 0)),
            pl.BlockSpec((_RB, 1), lambda i: (i, 0)),
        ],
        out_specs=[
            pl.BlockSpec((_RB, _D), lambda i: (i, 0)),
            pl.BlockSpec((_RB, 1), lambda i: (i, 0)),
        ],
        out_shape=[
            jax.ShapeDtypeStruct((_NP, _D), jnp.float32),
            jax.ShapeDtypeStruct((_NP, 1), jnp.float32),
        ],
    )(x_pad, w1, deg_col)


def _p2_body(s1_ref, p1_ref, dinv_ref, b1_ref, w2_ref, p2_ref):
    di = dinv_ref[...]
    h = di * (s1_ref[0] + s1_ref[1] + p1_ref[...]) + b1_ref[...]
    h = jnp.maximum(h, 0.0)
    p2_ref[...] = di * jnp.dot(h, w2_ref[...],
                               preferred_element_type=jnp.float32)


def _tc_p2(s1, p1, dinv, b1_row, w2p):
    grid = _NP // _RB
    return pl.pallas_call(
        _p2_body,
        grid=(grid,),
        in_specs=[
            pl.BlockSpec((2, _RB, _D), lambda i: (0, i, 0)),
            pl.BlockSpec((_RB, _D), lambda i: (i, 0)),
            pl.BlockSpec((_RB, 1), lambda i: (i, 0)),
            pl.BlockSpec((1, _D), lambda i: (0, 0)),
            pl.BlockSpec((_D, _CP), lambda i: (0, 0)),
        ],
        out_specs=pl.BlockSpec((_RB, _CP), lambda i: (i, 0)),
        out_shape=jax.ShapeDtypeStruct((_NP, _CP), jnp.float32),
    )(s1, p1, dinv, b1_row, w2p)


def _final_body(s2_ref, p2_ref, dinv_ref, b2_ref, logp_ref, logits_ref):
    di = dinv_ref[...]
    lg = di * (s2_ref[0] + s2_ref[1] + p2_ref[...]) + b2_ref[...]
    icol = lax.broadcasted_iota(jnp.int32, (_RB, _CP), 1)
    neg = jnp.float32(-jnp.inf)
    lm = jnp.where(icol < _C, lg, neg)
    m = jnp.max(lm, axis=1, keepdims=True)
    e = jnp.where(icol < _C, jnp.exp(lm - m), 0.0)
    lse = m + jnp.log(jnp.sum(e, axis=1, keepdims=True))
    logp_ref[...] = lg - lse
    logits_ref[...] = lg


def _tc_final(s2, p2, dinv, b2_row):
    grid = _NP // _RB
    return pl.pallas_call(
        _final_body,
        grid=(grid,),
        in_specs=[
            pl.BlockSpec((2, _RB, _CP), lambda i: (0, i, 0)),
            pl.BlockSpec((_RB, _CP), lambda i: (i, 0)),
            pl.BlockSpec((_RB, 1), lambda i: (i, 0)),
            pl.BlockSpec((1, _CP), lambda i: (0, 0)),
        ],
        out_specs=[
            pl.BlockSpec((_RB, _CP), lambda i: (i, 0)),
            pl.BlockSpec((_RB, _CP), lambda i: (i, 0)),
        ],
        out_shape=[
            jax.ShapeDtypeStruct((_NP, _CP), jnp.float32),
            jax.ShapeDtypeStruct((_NP, _CP), jnp.float32),
        ],
    )(s2, p2, dinv, b2_row)


def kernel(x, edge_index, W1, b1, W2, b2):
    row = edge_index[0].astype(jnp.int32)
    col = edge_index[1].astype(jnp.int32)

    degp = _degree(col)
    deg_col = (degp[:_NP] + degp[_NP:] + 1.0).reshape(_NP, 1)

    x_pad = jnp.pad(x, ((0, _NP - _N), (0, 0)))
    p1, dinv = _tc_p1(x_pad, W1, deg_col)

    s1 = _edge_scatter_d(p1, row, col).reshape(_NC, _NP, _D)

    b1_row = b1.reshape(1, _D)
    w2p = jnp.pad(W2, ((0, 0), (0, _CP - _C)))
    p2 = _tc_p2(s1, p1, dinv, b1_row, w2p)

    s2 = _edge_scatter_c(p2, row, col).reshape(_NC, _NP, _CP)

    b2_row = jnp.pad(b2, (0, _CP - _C)).reshape(1, _CP)
    logp, logits = _tc_final(s2, p2, dinv, b2_row)

    return (logp[:_N, :_C], logits[:_N, :_C])


# speedup vs baseline: 30.9962x; 2.3032x over previous
import functools

import jax
import jax.numpy as jnp
from jax import lax
from jax.experimental import pallas as pl
from jax.experimental.pallas import tpu as pltpu
from jax.experimental.pallas import tpu_sc as plsc

_N = 10000
_E = 320000
_D = 128
_C = 40
_CP = 64
_NPD = 10240
_NC = 2
_NS = 16
_NW = _NC * _NS
_EPW = _E // _NW
_K = 100
_NCHUNK = _EPW // _K
_STRIP = _N // _NS
_RB = 1000

_MESH = dict(core_axis_name="c", subcore_axis_name="s",
             num_cores=_NC, num_subcores=_NS)


def _zero_fill_2d(ref, nrows, dim):
    zeros16 = jnp.zeros((16,), jnp.float32)
    per_row = dim // 16

    def body(i, carry):
        ref[i // per_row, pl.ds((i % per_row) * 16, 16)] = zeros16
        return carry

    lax.fori_loop(0, nrows * per_row, body, 0)


def _zero_fill_1d(ref, n):
    zeros16 = jnp.zeros((16,), jnp.float32)

    def body(i, carry):
        ref[pl.ds(i * 16, 16)] = zeros16
        return carry

    lax.fori_loop(0, n // 16, body, 0)


def _make_edge_scatter(dim, nslot):
    mesh = plsc.VectorSubcoreMesh(**_MESH)

    @functools.partial(
        pl.kernel,
        out_type=jax.ShapeDtypeStruct((_NC * _N, dim), jnp.float32),
        mesh=mesh,
        scratch_types=[
            pltpu.VMEM((_NCHUNK, _K), jnp.int32),
            pltpu.VMEM((_NCHUNK, _K), jnp.int32),
            [pltpu.VMEM((_K, dim), jnp.float32) for _ in range(nslot)],
            pltpu.VMEM_SHARED((_N, dim), jnp.float32),
            [pltpu.SemaphoreType.DMA for _ in range(nslot)],
            [pltpu.SemaphoreType.DMA for _ in range(nslot)],
        ],
        compiler_params=pltpu.CompilerParams(use_tc_tiling_on_sc=False),
    )
    def scatter_kernel(p_hbm, row_hbm, col_hbm, out_hbm,
                       row_v, col_v, bufs, acc, gsems, ssems):
        c = lax.axis_index("c")
        s = lax.axis_index("s")
        wid = c * _NS + s
        strip = s * _STRIP

        pltpu.sync_copy(row_hbm.at[pl.ds(wid * _NCHUNK, _NCHUNK), :], row_v)
        pltpu.sync_copy(col_hbm.at[pl.ds(wid * _NCHUNK, _NCHUNK), :], col_v)
        _zero_fill_2d(bufs[0], _K, dim)
        for z in range(_STRIP // _K):
            pltpu.sync_copy(bufs[0], acc.at[pl.ds(strip + z * _K, _K), :])
        pltpu.sync_copy(bufs[0].at[pl.ds(0, _STRIP % _K), :],
                        acc.at[pl.ds(strip + _STRIP - _STRIP % _K,
                                     _STRIP % _K), :])
        plsc.subcore_barrier()

        for b in range(nslot):
            pltpu.async_copy(p_hbm.at[row_v.at[b]], bufs[b], gsems[b])

        def body(r, carry):
            handles = []
            for b in range(nslot):
                j = r * nslot + b
                pltpu.make_async_copy(p_hbm.at[row_v.at[j]], bufs[b],
                                      gsems[b]).wait()
                handles.append(pltpu.async_copy(
                    bufs[b], acc.at[col_v.at[j]], ssems[b], add=True))
            for b in range(nslot):
                handles[b].wait()
                j2 = r * nslot + b + nslot

                @pl.when(j2 < _NCHUNK)
                def _():
                    pltpu.async_copy(p_hbm.at[row_v.at[j2]], bufs[b],
                                     gsems[b])
            return carry

        lax.fori_loop(0, _NCHUNK // nslot, body, 0)
        plsc.subcore_barrier()

        nz = _STRIP // _K
        for z in range(nz):
            r0 = strip + z * _K
            pltpu.sync_copy(acc.at[pl.ds(r0, _K), :], bufs[0])
            pltpu.sync_copy(bufs[0], out_hbm.at[pl.ds(c * _N + r0, _K), :])
        rem = _STRIP % _K
        r0 = strip + nz * _K
        pltpu.sync_copy(acc.at[pl.ds(r0, rem), :],
                        bufs[0].at[pl.ds(0, rem), :])
        pltpu.sync_copy(bufs[0].at[pl.ds(0, rem), :],
                        out_hbm.at[pl.ds(c * _N + r0, rem), :])

    return scatter_kernel


def _make_degree():
    mesh = plsc.VectorSubcoreMesh(**_MESH)
    fire = 10

    @functools.partial(
        pl.kernel,
        out_type=jax.ShapeDtypeStruct((_NC * _NPD,), jnp.float32),
        mesh=mesh,
        scratch_types=[
            pltpu.VMEM((_NCHUNK, _K), jnp.int32),
            pltpu.VMEM((_K,), jnp.float32),
            pltpu.VMEM((_NPD // _NS,), jnp.float32),
            pltpu.VMEM_SHARED((_NPD,), jnp.float32),
            pltpu.SemaphoreType.DMA,
        ],
        compiler_params=pltpu.CompilerParams(use_tc_tiling_on_sc=False),
    )
    def degree_kernel(col_hbm, out_hbm, col_v, ones_v, stage_v, acc, sem):
        c = lax.axis_index("c")
        s = lax.axis_index("s")
        wid = c * _NS + s
        dstrip = _NPD // _NS
        strip = s * dstrip

        ones16 = jnp.ones((16,), jnp.float32)
        for i in range(_K // 16 + 1):
            ones_v[pl.ds(min(i * 16, _K - 16), 16)] = ones16
        pltpu.sync_copy(col_hbm.at[pl.ds(wid * _NCHUNK, _NCHUNK), :], col_v)
        _zero_fill_1d(stage_v, dstrip)
        pltpu.sync_copy(stage_v, acc.at[pl.ds(strip, dstrip)])
        plsc.subcore_barrier()

        def body(r, carry):
            handles = [
                pltpu.async_copy(ones_v, acc.at[col_v.at[r * fire + b]],
                                 sem, add=True)
                for b in range(fire)
            ]
            for h in handles:
                h.wait()
            return carry

        lax.fori_loop(0, _NCHUNK // fire, body, 0)
        plsc.subcore_barrier()

        pltpu.sync_copy(acc.at[pl.ds(strip, dstrip)], stage_v)
        pltpu.sync_copy(stage_v, out_hbm.at[pl.ds(c * _NPD + strip, dstrip)])

    return degree_kernel


_edge_scatter_d = _make_edge_scatter(_D, 2)
_edge_scatter_c = _make_edge_scatter(_CP, 4)
_degree = _make_degree()


def _p1_body(x_ref, w1_ref, deg_ref, p1_ref, dinv_ref):
    di = lax.rsqrt(deg_ref[...])
    p1_ref[...] = di * jnp.dot(x_ref[...], w1_ref[...],
                               preferred_element_type=jnp.float32)
    dinv_ref[...] = di


def _tc_p1(x, w1, deg_col):
    grid = _N // _RB
    return pl.pallas_call(
        _p1_body,
        grid=(grid,),
        in_specs=[
            pl.BlockSpec((_RB, _D), lambda i: (i, 0)),
            pl.BlockSpec((_D, _D), lambda i: (0, 0)),
            pl.BlockSpec((_RB, 1), lambda i: (i, 0)),
        ],
        out_specs=[
            pl.BlockSpec((_RB, _D), lambda i: (i, 0)),
            pl.BlockSpec((_RB, 1), lambda i: (i, 0)),
        ],
        out_shape=[
            jax.ShapeDtypeStruct((_N, _D), jnp.float32),
            jax.ShapeDtypeStruct((_N, 1), jnp.float32),
        ],
    )(x, w1, deg_col)


def _p2_body(s1_ref, p1_ref, dinv_ref, b1_ref, w2_ref, p2_ref):
    di = dinv_ref[...]
    h = di * (s1_ref[0] + s1_ref[1] + p1_ref[...]) + b1_ref[...]
    h = jnp.maximum(h, 0.0)
    p2_ref[...] = di * jnp.dot(h, w2_ref[...],
                               preferred_element_type=jnp.float32)


def _tc_p2(s1, p1, dinv, b1_row, w2p):
    grid = _N // _RB
    return pl.pallas_call(
        _p2_body,
        grid=(grid,),
        in_specs=[
            pl.BlockSpec((2, _RB, _D), lambda i: (0, i, 0)),
            pl.BlockSpec((_RB, _D), lambda i: (i, 0)),
            pl.BlockSpec((_RB, 1), lambda i: (i, 0)),
            pl.BlockSpec((1, _D), lambda i: (0, 0)),
            pl.BlockSpec((_D, _CP), lambda i: (0, 0)),
        ],
        out_specs=pl.BlockSpec((_RB, _CP), lambda i: (i, 0)),
        out_shape=jax.ShapeDtypeStruct((_N, _CP), jnp.float32),
    )(s1, p1, dinv, b1_row, w2p)


def _final_body(s2_ref, p2_ref, dinv_ref, b2_ref, logp_ref, logits_ref):
    di = dinv_ref[...]
    lg = di * (s2_ref[0] + s2_ref[1] + p2_ref[...]) + b2_ref[...]
    icol = lax.broadcasted_iota(jnp.int32, (_RB, _CP), 1)
    neg = jnp.float32(-jnp.inf)
    lm = jnp.where(icol < _C, lg, neg)
    m = jnp.max(lm, axis=1, keepdims=True)
    e = jnp.where(icol < _C, jnp.exp(lm - m), 0.0)
    lse = m + jnp.log(jnp.sum(e, axis=1, keepdims=True))
    logp_ref[...] = lg - lse
    logits_ref[...] = lg


def _tc_final(s2, p2, dinv, b2_row):
    grid = _N // _RB
    return pl.pallas_call(
        _final_body,
        grid=(grid,),
        in_specs=[
            pl.BlockSpec((2, _RB, _CP), lambda i: (0, i, 0)),
            pl.BlockSpec((_RB, _CP), lambda i: (i, 0)),
            pl.BlockSpec((_RB, 1), lambda i: (i, 0)),
            pl.BlockSpec((1, _CP), lambda i: (0, 0)),
        ],
        out_specs=[
            pl.BlockSpec((_RB, _CP), lambda i: (i, 0)),
            pl.BlockSpec((_RB, _CP), lambda i: (i, 0)),
        ],
        out_shape=[
            jax.ShapeDtypeStruct((_N, _CP), jnp.float32),
            jax.ShapeDtypeStruct((_N, _CP), jnp.float32),
        ],
    )(s2, p2, dinv, b2_row)


def kernel(x, edge_index, W1, b1, W2, b2):
    row = edge_index[0].astype(jnp.int32).reshape(_E // _K, _K)
    col = edge_index[1].astype(jnp.int32).reshape(_E // _K, _K)

    degp = _degree(col)
    deg_col = (degp[:_N] + degp[_NPD:_NPD + _N] + 1.0).reshape(_N, 1)

    p1, dinv = _tc_p1(x, W1, deg_col)

    s1 = _edge_scatter_d(p1, row, col).reshape(_NC, _N, _D)

    b1_row = b1.reshape(1, _D)
    w2p = jnp.pad(W2, ((0, 0), (0, _CP - _C)))
    p2 = _tc_p2(s1, p1, dinv, b1_row, w2p)

    s2 = _edge_scatter_c(p2, row, col).reshape(_NC, _N, _CP)

    b2_row = jnp.pad(b2, (0, _CP - _C)).reshape(1, _CP)
    logp, logits = _tc_final(s2, p2, dinv, b2_row)

    return (logp[:, :_C], logits[:, :_C])


# speedup vs baseline: 33.7493x; 1.0888x over previous
import functools

import jax
import jax.numpy as jnp
from jax import lax
from jax.experimental import pallas as pl
from jax.experimental.pallas import tpu as pltpu
from jax.experimental.pallas import tpu_sc as plsc

_N = 10000
_E = 320000
_D = 128
_C = 40
_CP = 64
_NPD = 10240
_NC = 2
_NS = 16
_NW = _NC * _NS
_K = 128
_NCH = _E // _K
_CPW = _NCH // _NW
_XTRA = _NCH - _CPW * _NW
_STRIP = _N // _NS
_RB = 2000

_MESH = dict(core_axis_name="c", subcore_axis_name="s",
             num_cores=_NC, num_subcores=_NS)


def _zero_fill_2d(ref, nrows, dim):
    zeros16 = jnp.zeros((16,), jnp.float32)
    per_row = dim // 16

    def body(i, carry):
        ref[i // per_row, pl.ds((i % per_row) * 16, 16)] = zeros16
        return carry

    lax.fori_loop(0, nrows * per_row, body, 0)


def _zero_fill_1d(ref, n):
    zeros16 = jnp.zeros((16,), jnp.float32)

    def body(i, carry):
        ref[pl.ds(i * 16, 16)] = zeros16
        return carry

    lax.fori_loop(0, n // 16, body, 0)


def _zero_acc_strip(zsrc, acc, strip, dim):
    nz = _STRIP // _K
    for z in range(nz):
        pltpu.sync_copy(zsrc, acc.at[pl.ds(strip + z * _K, _K), :])
    rem = _STRIP % _K
    if rem:
        pltpu.sync_copy(zsrc.at[pl.ds(0, rem), :],
                        acc.at[pl.ds(strip + nz * _K, rem), :])


def _write_out_strip(acc, out_hbm, c, strip, stage, dim):
    nz = _STRIP // _K
    for z in range(nz):
        r0 = strip + z * _K
        pltpu.sync_copy(acc.at[pl.ds(r0, _K), :], stage)
        pltpu.sync_copy(stage, out_hbm.at[c, pl.ds(r0, _K), :])
    rem = _STRIP % _K
    if rem:
        r0 = strip + nz * _K
        pltpu.sync_copy(acc.at[pl.ds(r0, rem), :],
                        stage.at[pl.ds(0, rem), :])
        pltpu.sync_copy(stage.at[pl.ds(0, rem), :],
                        out_hbm.at[c, pl.ds(r0, rem), :])


def _make_edge_scatter_grouped(dim, nslot, ngrp):
    assert _CPW % ngrp == 0
    grp = _CPW // ngrp
    assert grp % nslot == 0
    mesh = plsc.VectorSubcoreMesh(**_MESH)

    @functools.partial(
        pl.kernel,
        out_type=jax.ShapeDtypeStruct((_NC, _N, dim), jnp.float32),
        mesh=mesh,
        scratch_types=[
            [pltpu.VMEM((grp, _K), jnp.int32) for _ in range(2)],
            [pltpu.VMEM((grp, _K), jnp.int32) for _ in range(2)],
            [pltpu.VMEM((_K, dim), jnp.float32) for _ in range(nslot)],
            pltpu.VMEM_SHARED((_N, dim), jnp.float32),
            [pltpu.SemaphoreType.DMA for _ in range(2)],
            [pltpu.SemaphoreType.DMA for _ in range(nslot)],
            [pltpu.SemaphoreType.DMA for _ in range(nslot)],
        ],
        compiler_params=pltpu.CompilerParams(use_tc_tiling_on_sc=False),
    )
    def scatter_kernel(p_hbm, edge_hbm, out_hbm,
                       rg, cg, bufs, acc, isems, gsems, ssems):
        c = lax.axis_index("c")
        s = lax.axis_index("s")
        wid = c * _NS + s
        strip = s * _STRIP
        cbase = wid * _CPW

        def stage_group(g, gb):
            pltpu.async_copy(edge_hbm.at[0, pl.ds(cbase + g * grp, grp), :],
                             rg[gb], isems[gb])
            pltpu.async_copy(edge_hbm.at[1, pl.ds(cbase + g * grp, grp), :],
                             cg[gb], isems[gb])

        def wait_group(g, gb):
            pltpu.make_async_copy(
                edge_hbm.at[0, pl.ds(cbase + g * grp, grp), :],
                rg[gb], isems[gb]).wait()
            pltpu.make_async_copy(
                edge_hbm.at[1, pl.ds(cbase + g * grp, grp), :],
                cg[gb], isems[gb]).wait()

        stage_group(0, 0)
        _zero_fill_2d(bufs[0], _K, dim)
        _zero_acc_strip(bufs[0], acc, strip, dim)
        plsc.subcore_barrier()

        for g in range(ngrp):
            gb = g % 2
            wait_group(g, gb)
            if g + 1 < ngrp:
                stage_group(g + 1, (g + 1) % 2)
            for b in range(nslot):
                pltpu.async_copy(p_hbm.at[rg[gb].at[b]], bufs[b], gsems[b])

            def body(r, carry):
                handles = []
                for b in range(nslot):
                    j = r * nslot + b
                    pltpu.make_async_copy(p_hbm.at[rg[gb].at[j]], bufs[b],
                                          gsems[b]).wait()
                    handles.append(pltpu.async_copy(
                        bufs[b], acc.at[cg[gb].at[j]], ssems[b], add=True))
                for b in range(nslot):
                    handles[b].wait()
                    j2 = r * nslot + b + nslot

                    @pl.when(j2 < grp)
                    def _():
                        pltpu.async_copy(p_hbm.at[rg[gb].at[j2]], bufs[b],
                                         gsems[b])
                return carry

            lax.fori_loop(0, grp // nslot, body, 0)

        @pl.when(wid < _XTRA)
        def _():
            cid = _NW * _CPW + wid
            pltpu.sync_copy(edge_hbm.at[0, cid, :], rg[0].at[0])
            pltpu.sync_copy(edge_hbm.at[1, cid, :], cg[0].at[0])
            pltpu.async_copy(p_hbm.at[rg[0].at[0]], bufs[0],
                             gsems[0]).wait()
            pltpu.sync_copy(bufs[0], acc.at[cg[0].at[0]], add=True)

        plsc.subcore_barrier()
        _write_out_strip(acc, out_hbm, c, strip, bufs[0], dim)

    return scatter_kernel


def _make_edge_scatter_full(dim, nslot):
    assert _CPW % nslot == 0
    mesh = plsc.VectorSubcoreMesh(**_MESH)

    @functools.partial(
        pl.kernel,
        out_type=jax.ShapeDtypeStruct((_NC, _N, dim), jnp.float32),
        mesh=mesh,
        scratch_types=[
            pltpu.VMEM((_CPW + 1, _K), jnp.int32),
            pltpu.VMEM((_CPW + 1, _K), jnp.int32),
            [pltpu.VMEM((_K, dim), jnp.float32) for _ in range(nslot)],
            pltpu.VMEM_SHARED((_N, dim), jnp.float32),
            [pltpu.SemaphoreType.DMA for _ in range(nslot)],
            [pltpu.SemaphoreType.DMA for _ in range(nslot)],
        ],
        compiler_params=pltpu.CompilerParams(use_tc_tiling_on_sc=False),
    )
    def scatter_kernel(p_hbm, edge_hbm, out_hbm,
                       row_v, col_v, bufs, acc, gsems, ssems):
        c = lax.axis_index("c")
        s = lax.axis_index("s")
        wid = c * _NS + s
        strip = s * _STRIP
        cbase = wid * _CPW

        pltpu.sync_copy(edge_hbm.at[0, pl.ds(cbase, _CPW), :],
                        row_v.at[pl.ds(0, _CPW), :])
        pltpu.sync_copy(edge_hbm.at[1, pl.ds(cbase, _CPW), :],
                        col_v.at[pl.ds(0, _CPW), :])
        _zero_fill_2d(bufs[0], _K, dim)
        _zero_acc_strip(bufs[0], acc, strip, dim)
        plsc.subcore_barrier()

        for b in range(nslot):
            pltpu.async_copy(p_hbm.at[row_v.at[b]], bufs[b], gsems[b])

        def body(r, carry):
            handles = []
            for b in range(nslot):
                j = r * nslot + b
                pltpu.make_async_copy(p_hbm.at[row_v.at[j]], bufs[b],
                                      gsems[b]).wait()
                handles.append(pltpu.async_copy(
                    bufs[b], acc.at[col_v.at[j]], ssems[b], add=True))
            for b in range(nslot):
                handles[b].wait()
                j2 = r * nslot + b + nslot

                @pl.when(j2 < _CPW)
                def _():
                    pltpu.async_copy(p_hbm.at[row_v.at[j2]], bufs[b],
                                     gsems[b])
            return carry

        lax.fori_loop(0, _CPW // nslot, body, 0)

        @pl.when(wid < _XTRA)
        def _():
            cid = _NW * _CPW + wid
            pltpu.sync_copy(edge_hbm.at[0, cid, :], row_v.at[_CPW])
            pltpu.sync_copy(edge_hbm.at[1, cid, :], col_v.at[_CPW])
            pltpu.async_copy(p_hbm.at[row_v.at[_CPW]], bufs[0],
                             gsems[0]).wait()
            pltpu.sync_copy(bufs[0], acc.at[col_v.at[_CPW]], add=True)

        plsc.subcore_barrier()
        _write_out_strip(acc, out_hbm, c, strip, bufs[0], dim)

    return scatter_kernel


def _make_degree():
    mesh = plsc.VectorSubcoreMesh(**_MESH)
    fire = 6

    @functools.partial(
        pl.kernel,
        out_type=jax.ShapeDtypeStruct((_NC * _NPD,), jnp.float32),
        mesh=mesh,
        scratch_types=[
            pltpu.VMEM((_CPW + 1, _K), jnp.int32),
            pltpu.VMEM((_K,), jnp.float32),
            pltpu.VMEM((_NPD // _NS,), jnp.float32),
            pltpu.VMEM_SHARED((_NPD,), jnp.float32),
            pltpu.SemaphoreType.DMA,
        ],
        compiler_params=pltpu.CompilerParams(use_tc_tiling_on_sc=False),
    )
    def degree_kernel(edge_hbm, out_hbm, col_v, ones_v, stage_v, acc, sem):
        c = lax.axis_index("c")
        s = lax.axis_index("s")
        wid = c * _NS + s
        dstrip = _NPD // _NS
        strip = s * dstrip

        ones16 = jnp.ones((16,), jnp.float32)
        for i in range(_K // 16):
            ones_v[pl.ds(i * 16, 16)] = ones16
        pltpu.sync_copy(edge_hbm.at[1, pl.ds(wid * _CPW, _CPW), :],
                        col_v.at[pl.ds(0, _CPW), :])
        _zero_fill_1d(stage_v, dstrip)
        pltpu.sync_copy(stage_v, acc.at[pl.ds(strip, dstrip)])
        plsc.subcore_barrier()

        def body(r, carry):
            handles = [
                pltpu.async_copy(ones_v, acc.at[col_v.at[r * fire + b]],
                                 sem, add=True)
                for b in range(fire)
            ]
            for h in handles:
                h.wait()
            return carry

        lax.fori_loop(0, _CPW // fire, body, 0)

        @pl.when(wid < _XTRA)
        def _():
            cid = _NW * _CPW + wid
            pltpu.sync_copy(edge_hbm.at[1, cid, :], col_v.at[_CPW])
            pltpu.sync_copy(ones_v, acc.at[col_v.at[_CPW]], add=True)

        plsc.subcore_barrier()
        pltpu.sync_copy(acc.at[pl.ds(strip, dstrip)], stage_v)
        pltpu.sync_copy(stage_v, out_hbm.at[pl.ds(c * _NPD + strip, dstrip)])

    return degree_kernel


_edge_scatter_d = _make_edge_scatter_grouped(_D, 2, 3)
_edge_scatter_c = _make_edge_scatter_full(_CP, 6)
_degree = _make_degree()


def _p1_body(x_ref, w1_ref, deg_ref, p1_ref, dinv_ref):
    di = lax.rsqrt(deg_ref[...])
    p1_ref[...] = di * jnp.dot(x_ref[...], w1_ref[...],
                               preferred_element_type=jnp.float32)
    dinv_ref[...] = di


def _tc_p1(x, w1, deg_col):
    grid = _N // _RB
    return pl.pallas_call(
        _p1_body,
        grid=(grid,),
        in_specs=[
            pl.BlockSpec((_RB, _D), lambda i: (i, 0)),
            pl.BlockSpec((_D, _D), lambda i: (0, 0)),
            pl.BlockSpec((_RB, 1), lambda i: (i, 0)),
        ],
        out_specs=[
            pl.BlockSpec((_RB, _D), lambda i: (i, 0)),
            pl.BlockSpec((_RB, 1), lambda i: (i, 0)),
        ],
        out_shape=[
            jax.ShapeDtypeStruct((_N, _D), jnp.float32),
            jax.ShapeDtypeStruct((_N, 1), jnp.float32),
        ],
    )(x, w1, deg_col)


def _p2_body(s1_ref, p1_ref, dinv_ref, b1_ref, w2_ref, p2_ref):
    di = dinv_ref[...]
    h = di * (s1_ref[0] + s1_ref[1] + p1_ref[...]) + b1_ref[...]
    h = jnp.maximum(h, 0.0)
    p2_ref[...] = di * jnp.dot(h, w2_ref[...],
                               preferred_element_type=jnp.float32)


def _tc_p2(s1, p1, dinv, b1_row, w2p):
    grid = _N // _RB
    return pl.pallas_call(
        _p2_body,
        grid=(grid,),
        in_specs=[
            pl.BlockSpec((2, _RB, _D), lambda i: (0, i, 0)),
            pl.BlockSpec((_RB, _D), lambda i: (i, 0)),
            pl.BlockSpec((_RB, 1), lambda i: (i, 0)),
            pl.BlockSpec((1, _D), lambda i: (0, 0)),
            pl.BlockSpec((_D, _CP), lambda i: (0, 0)),
        ],
        out_specs=pl.BlockSpec((_RB, _CP), lambda i: (i, 0)),
        out_shape=jax.ShapeDtypeStruct((_N, _CP), jnp.float32),
    )(s1, p1, dinv, b1_row, w2p)


def _final_body(s2_ref, p2_ref, dinv_ref, b2_ref, logp_ref, logits_ref):
    di = dinv_ref[...]
    lg = di * (s2_ref[0] + s2_ref[1] + p2_ref[...]) + b2_ref[...]
    icol = lax.broadcasted_iota(jnp.int32, (_RB, _CP), 1)
    neg = jnp.float32(-jnp.inf)
    lm = jnp.where(icol < _C, lg, neg)
    m = jnp.max(lm, axis=1, keepdims=True)
    e = jnp.where(icol < _C, jnp.exp(lm - m), 0.0)
    lse = m + jnp.log(jnp.sum(e, axis=1, keepdims=True))
    logp_ref[...] = (lg - lse)[:, :_C]
    logits_ref[...] = lg[:, :_C]


def _tc_final(s2, p2, dinv, b2_row):
    grid = _N // _RB
    return pl.pallas_call(
        _final_body,
        grid=(grid,),
        in_specs=[
            pl.BlockSpec((2, _RB, _CP), lambda i: (0, i, 0)),
            pl.BlockSpec((_RB, _CP), lambda i: (i, 0)),
            pl.BlockSpec((_RB, 1), lambda i: (i, 0)),
            pl.BlockSpec((1, _CP), lambda i: (0, 0)),
        ],
        out_specs=[
            pl.BlockSpec((_RB, _C), lambda i: (i, 0)),
            pl.BlockSpec((_RB, _C), lambda i: (i, 0)),
        ],
        out_shape=[
            jax.ShapeDtypeStruct((_N, _C), jnp.float32),
            jax.ShapeDtypeStruct((_N, _C), jnp.float32),
        ],
    )(s2, p2, dinv, b2_row)


def kernel(x, edge_index, W1, b1, W2, b2):
    edges = edge_index.astype(jnp.int32).reshape(2, _NCH, _K)

    degp = _degree(edges)
    deg_col = (degp[:_N] + degp[_NPD:_NPD + _N] + 1.0).reshape(_N, 1)

    p1, dinv = _tc_p1(x, W1, deg_col)

    s1 = _edge_scatter_d(p1, edges)

    b1_row = b1.reshape(1, _D)
    w2p = jnp.pad(W2, ((0, 0), (0, _CP - _C)))
    p2 = _tc_p2(s1, p1, dinv, b1_row, w2p)

    s2 = _edge_scatter_c(p2, edges)

    b2_row = jnp.pad(b2, (0, _CP - _C)).reshape(1, _CP)
    logp, logits = _tc_final(s2, p2, dinv, b2_row)

    return (logp, logits)


# speedup vs baseline: 33.9480x; 1.0059x over previous
import functools

import jax
import jax.numpy as jnp
from jax import lax
from jax.experimental import pallas as pl
from jax.experimental.pallas import tpu as pltpu
from jax.experimental.pallas import tpu_sc as plsc

_N = 10000
_E = 320000
_D = 128
_C = 40
_CP = 64
_NPD = 10240
_NC = 2
_NS = 16
_NW = _NC * _NS
_K = 128
_NCH = _E // _K
_CPW = _NCH // _NW
_XTRA = _NCH - _CPW * _NW
_STRIP = _N // _NS
_RB = 2000

_MESH = dict(core_axis_name="c", subcore_axis_name="s",
             num_cores=_NC, num_subcores=_NS)


def _zero_fill_2d(ref, nrows, dim):
    zeros16 = jnp.zeros((16,), jnp.float32)
    per_row = dim // 16

    def body(i, carry):
        ref[i // per_row, pl.ds((i % per_row) * 16, 16)] = zeros16
        return carry

    lax.fori_loop(0, nrows * per_row, body, 0)


def _zero_fill_1d(ref, n):
    zeros16 = jnp.zeros((16,), jnp.float32)

    def body(i, carry):
        ref[pl.ds(i * 16, 16)] = zeros16
        return carry

    lax.fori_loop(0, n // 16, body, 0)


def _zero_acc_strip(zsrc, acc, strip, dim):
    nz = _STRIP // _K
    for z in range(nz):
        pltpu.sync_copy(zsrc, acc.at[pl.ds(strip + z * _K, _K), :])
    rem = _STRIP % _K
    if rem:
        pltpu.sync_copy(zsrc.at[pl.ds(0, rem), :],
                        acc.at[pl.ds(strip + nz * _K, rem), :])


def _write_out_strip(acc, out_hbm, c, strip, stage, dim):
    nz = _STRIP // _K
    for z in range(nz):
        r0 = strip + z * _K
        pltpu.sync_copy(acc.at[pl.ds(r0, _K), :], stage)
        pltpu.sync_copy(stage, out_hbm.at[c, pl.ds(r0, _K), :])
    rem = _STRIP % _K
    if rem:
        r0 = strip + nz * _K
        pltpu.sync_copy(acc.at[pl.ds(r0, rem), :],
                        stage.at[pl.ds(0, rem), :])
        pltpu.sync_copy(stage.at[pl.ds(0, rem), :],
                        out_hbm.at[c, pl.ds(r0, rem), :])


def _make_edge_scatter_jit(dim, nslot):
    assert _CPW % nslot == 0
    mesh = plsc.VectorSubcoreMesh(**_MESH)

    @functools.partial(
        pl.kernel,
        out_type=jax.ShapeDtypeStruct((_NC, _N, dim), jnp.float32),
        mesh=mesh,
        scratch_types=[
            [pltpu.VMEM((_K,), jnp.int32) for _ in range(nslot)],
            [pltpu.VMEM((_K,), jnp.int32) for _ in range(nslot)],
            [pltpu.VMEM((_K, dim), jnp.float32) for _ in range(nslot)],
            pltpu.VMEM_SHARED((_N, dim), jnp.float32),
            [pltpu.SemaphoreType.DMA for _ in range(nslot)],
            [pltpu.SemaphoreType.DMA for _ in range(nslot)],
            [pltpu.SemaphoreType.DMA for _ in range(nslot)],
        ],
        compiler_params=pltpu.CompilerParams(use_tc_tiling_on_sc=False),
    )
    def scatter_kernel(p_hbm, edge_hbm, out_hbm,
                       rowb, colb, bufs, acc, isems, gsems, ssems):
        c = lax.axis_index("c")
        s = lax.axis_index("s")
        wid = c * _NS + s
        strip = s * _STRIP
        cbase = wid * _CPW

        def stage_idx(j, b):
            pltpu.async_copy(edge_hbm.at[0, cbase + j, :], rowb[b], isems[b])
            pltpu.async_copy(edge_hbm.at[1, cbase + j, :], colb[b], isems[b])

        def wait_idx(j, b):
            pltpu.make_async_copy(edge_hbm.at[0, cbase + j, :], rowb[b],
                                  isems[b]).wait()
            pltpu.make_async_copy(edge_hbm.at[1, cbase + j, :], colb[b],
                                  isems[b]).wait()

        for b in range(nslot):
            stage_idx(b, b)
        _zero_fill_2d(bufs[0], _K, dim)
        _zero_acc_strip(bufs[0], acc, strip, dim)
        plsc.subcore_barrier()

        def body(r, carry):
            gh = []
            for b in range(nslot):
                j = r * nslot + b
                wait_idx(j, b)
                gh.append(pltpu.async_copy(p_hbm.at[rowb[b]], bufs[b],
                                           gsems[b]))
            sh = []
            for b in range(nslot):
                gh[b].wait()
                sh.append(pltpu.async_copy(bufs[b], acc.at[colb[b]],
                                           ssems[b], add=True))
            for b in range(nslot):
                sh[b].wait()
                j2 = r * nslot + b + nslot

                @pl.when(j2 < _CPW)
                def _():
                    stage_idx(j2, b)
            return carry

        lax.fori_loop(0, _CPW // nslot, body, 0)

        @pl.when(wid < _XTRA)
        def _():
            cid = _NW * _CPW + wid
            pltpu.sync_copy(edge_hbm.at[0, cid, :], rowb[0])
            pltpu.sync_copy(edge_hbm.at[1, cid, :], colb[0])
            pltpu.async_copy(p_hbm.at[rowb[0]], bufs[0], gsems[0]).wait()
            pltpu.sync_copy(bufs[0], acc.at[colb[0]], add=True)

        plsc.subcore_barrier()
        _write_out_strip(acc, out_hbm, c, strip, bufs[0], dim)

    return scatter_kernel


def _make_edge_scatter_full(dim, nslot):
    assert _CPW % nslot == 0
    mesh = plsc.VectorSubcoreMesh(**_MESH)

    @functools.partial(
        pl.kernel,
        out_type=jax.ShapeDtypeStruct((_NC, _N, dim), jnp.float32),
        mesh=mesh,
        scratch_types=[
            pltpu.VMEM((_CPW + 1, _K), jnp.int32),
            pltpu.VMEM((_CPW + 1, _K), jnp.int32),
            [pltpu.VMEM((_K, dim), jnp.float32) for _ in range(nslot)],
            pltpu.VMEM_SHARED((_N, dim), jnp.float32),
            [pltpu.SemaphoreType.DMA for _ in range(nslot)],
            [pltpu.SemaphoreType.DMA for _ in range(nslot)],
        ],
        compiler_params=pltpu.CompilerParams(use_tc_tiling_on_sc=False),
    )
    def scatter_kernel(p_hbm, edge_hbm, out_hbm,
                       row_v, col_v, bufs, acc, gsems, ssems):
        c = lax.axis_index("c")
        s = lax.axis_index("s")
        wid = c * _NS + s
        strip = s * _STRIP
        cbase = wid * _CPW

        pltpu.sync_copy(edge_hbm.at[0, pl.ds(cbase, _CPW), :],
                        row_v.at[pl.ds(0, _CPW), :])
        pltpu.sync_copy(edge_hbm.at[1, pl.ds(cbase, _CPW), :],
                        col_v.at[pl.ds(0, _CPW), :])
        _zero_fill_2d(bufs[0], _K, dim)
        _zero_acc_strip(bufs[0], acc, strip, dim)
        plsc.subcore_barrier()

        for b in range(nslot):
            pltpu.async_copy(p_hbm.at[row_v.at[b]], bufs[b], gsems[b])

        def body(r, carry):
            handles = []
            for b in range(nslot):
                j = r * nslot + b
                pltpu.make_async_copy(p_hbm.at[row_v.at[j]], bufs[b],
                                      gsems[b]).wait()
                handles.append(pltpu.async_copy(
                    bufs[b], acc.at[col_v.at[j]], ssems[b], add=True))
            for b in range(nslot):
                handles[b].wait()
                j2 = r * nslot + b + nslot

                @pl.when(j2 < _CPW)
                def _():
                    pltpu.async_copy(p_hbm.at[row_v.at[j2]], bufs[b],
                                     gsems[b])
            return carry

        lax.fori_loop(0, _CPW // nslot, body, 0)

        @pl.when(wid < _XTRA)
        def _():
            cid = _NW * _CPW + wid
            pltpu.sync_copy(edge_hbm.at[0, cid, :], row_v.at[_CPW])
            pltpu.sync_copy(edge_hbm.at[1, cid, :], col_v.at[_CPW])
            pltpu.async_copy(p_hbm.at[row_v.at[_CPW]], bufs[0],
                             gsems[0]).wait()
            pltpu.sync_copy(bufs[0], acc.at[col_v.at[_CPW]], add=True)

        plsc.subcore_barrier()
        _write_out_strip(acc, out_hbm, c, strip, bufs[0], dim)

    return scatter_kernel


def _make_degree():
    mesh = plsc.VectorSubcoreMesh(**_MESH)
    fire = 6

    @functools.partial(
        pl.kernel,
        out_type=jax.ShapeDtypeStruct((_NC * _NPD,), jnp.float32),
        mesh=mesh,
        scratch_types=[
            pltpu.VMEM((_CPW + 1, _K), jnp.int32),
            pltpu.VMEM((_K,), jnp.float32),
            pltpu.VMEM((_NPD // _NS,), jnp.float32),
            pltpu.VMEM_SHARED((_NPD,), jnp.float32),
            pltpu.SemaphoreType.DMA,
        ],
        compiler_params=pltpu.CompilerParams(use_tc_tiling_on_sc=False),
    )
    def degree_kernel(edge_hbm, out_hbm, col_v, ones_v, stage_v, acc, sem):
        c = lax.axis_index("c")
        s = lax.axis_index("s")
        wid = c * _NS + s
        dstrip = _NPD // _NS
        strip = s * dstrip

        ones16 = jnp.ones((16,), jnp.float32)
        for i in range(_K // 16):
            ones_v[pl.ds(i * 16, 16)] = ones16
        pltpu.sync_copy(edge_hbm.at[1, pl.ds(wid * _CPW, _CPW), :],
                        col_v.at[pl.ds(0, _CPW), :])
        _zero_fill_1d(stage_v, dstrip)
        pltpu.sync_copy(stage_v, acc.at[pl.ds(strip, dstrip)])
        plsc.subcore_barrier()

        def body(r, carry):
            handles = [
                pltpu.async_copy(ones_v, acc.at[col_v.at[r * fire + b]],
                                 sem, add=True)
                for b in range(fire)
            ]
            for h in handles:
                h.wait()
            return carry

        lax.fori_loop(0, _CPW // fire, body, 0)

        @pl.when(wid < _XTRA)
        def _():
            cid = _NW * _CPW + wid
            pltpu.sync_copy(edge_hbm.at[1, cid, :], col_v.at[_CPW])
            pltpu.sync_copy(ones_v, acc.at[col_v.at[_CPW]], add=True)

        plsc.subcore_barrier()
        pltpu.sync_copy(acc.at[pl.ds(strip, dstrip)], stage_v)
        pltpu.sync_copy(stage_v, out_hbm.at[pl.ds(c * _NPD + strip, dstrip)])

    return degree_kernel


_edge_scatter_d = _make_edge_scatter_jit(_D, 3)
_edge_scatter_c = _make_edge_scatter_full(_CP, 6)
_degree = _make_degree()


def _p1_body(x_ref, w1_ref, deg_ref, p1_ref, dinv_ref):
    di = lax.rsqrt(deg_ref[...])
    p1_ref[...] = di * jnp.dot(x_ref[...], w1_ref[...],
                               preferred_element_type=jnp.float32)
    dinv_ref[...] = di


def _tc_p1(x, w1, deg_col):
    grid = _N // _RB
    return pl.pallas_call(
        _p1_body,
        grid=(grid,),
        in_specs=[
            pl.BlockSpec((_RB, _D), lambda i: (i, 0)),
            pl.BlockSpec((_D, _D), lambda i: (0, 0)),
            pl.BlockSpec((_RB, 1), lambda i: (i, 0)),
        ],
        out_specs=[
            pl.BlockSpec((_RB, _D), lambda i: (i, 0)),
            pl.BlockSpec((_RB, 1), lambda i: (i, 0)),
        ],
        out_shape=[
            jax.ShapeDtypeStruct((_N, _D), jnp.float32),
            jax.ShapeDtypeStruct((_N, 1), jnp.float32),
        ],
    )(x, w1, deg_col)


def _p2_body(s1_ref, p1_ref, dinv_ref, b1_ref, w2_ref, p2_ref):
    di = dinv_ref[...]
    h = di * (s1_ref[0] + s1_ref[1] + p1_ref[...]) + b1_ref[...]
    h = jnp.maximum(h, 0.0)
    p2_ref[...] = di * jnp.dot(h, w2_ref[...],
                               preferred_element_type=jnp.float32)


def _tc_p2(s1, p1, dinv, b1_row, w2p):
    grid = _N // _RB
    return pl.pallas_call(
        _p2_body,
        grid=(grid,),
        in_specs=[
            pl.BlockSpec((2, _RB, _D), lambda i: (0, i, 0)),
            pl.BlockSpec((_RB, _D), lambda i: (i, 0)),
            pl.BlockSpec((_RB, 1), lambda i: (i, 0)),
            pl.BlockSpec((1, _D), lambda i: (0, 0)),
            pl.BlockSpec((_D, _CP), lambda i: (0, 0)),
        ],
        out_specs=pl.BlockSpec((_RB, _CP), lambda i: (i, 0)),
        out_shape=jax.ShapeDtypeStruct((_N, _CP), jnp.float32),
    )(s1, p1, dinv, b1_row, w2p)


def _final_body(s2_ref, p2_ref, dinv_ref, b2_ref, logp_ref, logits_ref):
    di = dinv_ref[...]
    lg = di * (s2_ref[0] + s2_ref[1] + p2_ref[...]) + b2_ref[...]
    icol = lax.broadcasted_iota(jnp.int32, (_RB, _CP), 1)
    neg = jnp.float32(-jnp.inf)
    lm = jnp.where(icol < _C, lg, neg)
    m = jnp.max(lm, axis=1, keepdims=True)
    e = jnp.where(icol < _C, jnp.exp(lm - m), 0.0)
    lse = m + jnp.log(jnp.sum(e, axis=1, keepdims=True))
    logp_ref[...] = (lg - lse)[:, :_C]
    logits_ref[...] = lg[:, :_C]


def _tc_final(s2, p2, dinv, b2_row):
    grid = _N // _RB
    return pl.pallas_call(
        _final_body,
        grid=(grid,),
        in_specs=[
            pl.BlockSpec((2, _RB, _CP), lambda i: (0, i, 0)),
            pl.BlockSpec((_RB, _CP), lambda i: (i, 0)),
            pl.BlockSpec((_RB, 1), lambda i: (i, 0)),
            pl.BlockSpec((1, _CP), lambda i: (0, 0)),
        ],
        out_specs=[
            pl.BlockSpec((_RB, _C), lambda i: (i, 0)),
            pl.BlockSpec((_RB, _C), lambda i: (i, 0)),
        ],
        out_shape=[
            jax.ShapeDtypeStruct((_N, _C), jnp.float32),
            jax.ShapeDtypeStruct((_N, _C), jnp.float32),
        ],
    )(s2, p2, dinv, b2_row)


def kernel(x, edge_index, W1, b1, W2, b2):
    edges = edge_index.astype(jnp.int32).reshape(2, _NCH, _K)

    degp = _degree(edges)
    deg_col = (degp[:_N] + degp[_NPD:_NPD + _N] + 1.0).reshape(_N, 1)

    p1, dinv = _tc_p1(x, W1, deg_col)

    s1 = _edge_scatter_d(p1, edges)

    b1_row = b1.reshape(1, _D)
    w2p = jnp.pad(W2, ((0, 0), (0, _CP - _C)))
    p2 = _tc_p2(s1, p1, dinv, b1_row, w2p)

    s2 = _edge_scatter_c(p2, edges)

    b2_row = jnp.pad(b2, (0, _CP - _C)).reshape(1, _CP)
    logp, logits = _tc_final(s2, p2, dinv, b2_row)

    return (logp, logits)


# speedup vs baseline: 44.8306x; 1.3206x over previous
import functools

import jax
import jax.numpy as jnp
from jax import lax
from jax.experimental import pallas as pl
from jax.experimental.pallas import tpu as pltpu
from jax.experimental.pallas import tpu_sc as plsc

_N = 10000
_E = 320000
_D = 128
_C = 40
_CP = 64
_NPD = 10240
_NC = 2
_NS = 16
_NW = _NC * _NS
_K = 128
_NCH = _E // _K
_CPW = _NCH // _NW
_XTRA = _NCH - _CPW * _NW
_STRIP = _N // _NS
_RB = 2000

_MESH = dict(core_axis_name="c", subcore_axis_name="s",
             num_cores=_NC, num_subcores=_NS)


def _zero_fill_2d(ref, nrows, dim):
    zeros32 = jnp.zeros((32,), jnp.bfloat16)
    per_row = dim // 32

    def body(i, carry):
        ref[i // per_row, pl.ds((i % per_row) * 32, 32)] = zeros32
        return carry

    lax.fori_loop(0, nrows * per_row, body, 0)


def _zero_fill_1d(ref, n):
    zeros16 = jnp.zeros((16,), jnp.float32)

    def body(i, carry):
        ref[pl.ds(i * 16, 16)] = zeros16
        return carry

    lax.fori_loop(0, n // 16, body, 0)


def _zero_acc_strip(zsrc, acc, strip, dim):
    nz = _STRIP // _K
    for z in range(nz):
        pltpu.sync_copy(zsrc, acc.at[pl.ds(strip + z * _K, _K), :])
    rem = _STRIP % _K
    if rem:
        pltpu.sync_copy(zsrc.at[pl.ds(0, rem), :],
                        acc.at[pl.ds(strip + nz * _K, rem), :])


def _write_out_strip(acc, out_hbm, c, strip, stage, dim):
    nz = _STRIP // _K
    for z in range(nz):
        r0 = strip + z * _K
        pltpu.sync_copy(acc.at[pl.ds(r0, _K), :], stage)
        pltpu.sync_copy(stage, out_hbm.at[c, pl.ds(r0, _K), :])
    rem = _STRIP % _K
    if rem:
        r0 = strip + nz * _K
        pltpu.sync_copy(acc.at[pl.ds(r0, rem), :],
                        stage.at[pl.ds(0, rem), :])
        pltpu.sync_copy(stage.at[pl.ds(0, rem), :],
                        out_hbm.at[c, pl.ds(r0, rem), :])


def _make_edge_scatter_full(dim, nslot):
    assert _CPW % nslot == 0
    mesh = plsc.VectorSubcoreMesh(**_MESH)

    @functools.partial(
        pl.kernel,
        out_type=jax.ShapeDtypeStruct((_NC, _N, dim), jnp.bfloat16),
        mesh=mesh,
        scratch_types=[
            pltpu.VMEM((_CPW + 1, _K), jnp.int32),
            pltpu.VMEM((_CPW + 1, _K), jnp.int32),
            [pltpu.VMEM((_K, dim), jnp.bfloat16) for _ in range(nslot)],
            pltpu.VMEM_SHARED((_N, dim), jnp.bfloat16),
            [pltpu.SemaphoreType.DMA for _ in range(nslot)],
            [pltpu.SemaphoreType.DMA for _ in range(nslot)],
        ],
        compiler_params=pltpu.CompilerParams(use_tc_tiling_on_sc=False),
    )
    def scatter_kernel(p_hbm, edge_hbm, out_hbm,
                       row_v, col_v, bufs, acc, gsems, ssems):
        c = lax.axis_index("c")
        s = lax.axis_index("s")
        wid = c * _NS + s
        strip = s * _STRIP
        cbase = wid * _CPW

        pltpu.sync_copy(edge_hbm.at[0, pl.ds(cbase, _CPW), :],
                        row_v.at[pl.ds(0, _CPW), :])
        pltpu.sync_copy(edge_hbm.at[1, pl.ds(cbase, _CPW), :],
                        col_v.at[pl.ds(0, _CPW), :])
        _zero_fill_2d(bufs[0], _K, dim)
        _zero_acc_strip(bufs[0], acc, strip, dim)
        plsc.subcore_barrier()

        for b in range(nslot):
            pltpu.async_copy(p_hbm.at[row_v.at[b]], bufs[b], gsems[b])

        def body(r, carry):
            handles = []
            for b in range(nslot):
                j = r * nslot + b
                pltpu.make_async_copy(p_hbm.at[row_v.at[j]], bufs[b],
                                      gsems[b]).wait()
                handles.append(pltpu.async_copy(
                    bufs[b], acc.at[col_v.at[j]], ssems[b], add=True))
            for b in range(nslot):
                handles[b].wait()
                j2 = r * nslot + b + nslot

                @pl.when(j2 < _CPW)
                def _():
                    pltpu.async_copy(p_hbm.at[row_v.at[j2]], bufs[b],
                                     gsems[b])
            return carry

        lax.fori_loop(0, _CPW // nslot, body, 0)

        @pl.when(wid < _XTRA)
        def _():
            cid = _NW * _CPW + wid
            pltpu.sync_copy(edge_hbm.at[0, cid, :], row_v.at[_CPW])
            pltpu.sync_copy(edge_hbm.at[1, cid, :], col_v.at[_CPW])
            pltpu.async_copy(p_hbm.at[row_v.at[_CPW]], bufs[0],
                             gsems[0]).wait()
            pltpu.sync_copy(bufs[0], acc.at[col_v.at[_CPW]], add=True)

        plsc.subcore_barrier()
        _write_out_strip(acc, out_hbm, c, strip, bufs[0], dim)

    return scatter_kernel


def _make_degree():
    mesh = plsc.VectorSubcoreMesh(**_MESH)
    fire = 6

    @functools.partial(
        pl.kernel,
        out_type=jax.ShapeDtypeStruct((_NC * _NPD,), jnp.float32),
        mesh=mesh,
        scratch_types=[
            pltpu.VMEM((_CPW + 1, _K), jnp.int32),
            pltpu.VMEM((_K,), jnp.float32),
            pltpu.VMEM((_NPD // _NS,), jnp.float32),
            pltpu.VMEM_SHARED((_NPD,), jnp.float32),
            pltpu.SemaphoreType.DMA,
        ],
        compiler_params=pltpu.CompilerParams(use_tc_tiling_on_sc=False),
    )
    def degree_kernel(edge_hbm, out_hbm, col_v, ones_v, stage_v, acc, sem):
        c = lax.axis_index("c")
        s = lax.axis_index("s")
        wid = c * _NS + s
        dstrip = _NPD // _NS
        strip = s * dstrip

        ones16 = jnp.ones((16,), jnp.float32)
        for i in range(_K // 16):
            ones_v[pl.ds(i * 16, 16)] = ones16
        pltpu.sync_copy(edge_hbm.at[1, pl.ds(wid * _CPW, _CPW), :],
                        col_v.at[pl.ds(0, _CPW), :])
        _zero_fill_1d(stage_v, dstrip)
        pltpu.sync_copy(stage_v, acc.at[pl.ds(strip, dstrip)])
        plsc.subcore_barrier()

        def body(r, carry):
            handles = [
                pltpu.async_copy(ones_v, acc.at[col_v.at[r * fire + b]],
                                 sem, add=True)
                for b in range(fire)
            ]
            for h in handles:
                h.wait()
            return carry

        lax.fori_loop(0, _CPW // fire, body, 0)

        @pl.when(wid < _XTRA)
        def _():
            cid = _NW * _CPW + wid
            pltpu.sync_copy(edge_hbm.at[1, cid, :], col_v.at[_CPW])
            pltpu.sync_copy(ones_v, acc.at[col_v.at[_CPW]], add=True)

        plsc.subcore_barrier()
        pltpu.sync_copy(acc.at[pl.ds(strip, dstrip)], stage_v)
        pltpu.sync_copy(stage_v, out_hbm.at[pl.ds(c * _NPD + strip, dstrip)])

    return degree_kernel


_edge_scatter_d = _make_edge_scatter_full(_D, 6)
_edge_scatter_c = _make_edge_scatter_full(_CP, 6)
_degree = _make_degree()


def _p1_body(x_ref, w1_ref, deg_ref, p1_ref, p1h_ref, dinv_ref):
    di = lax.rsqrt(deg_ref[...])
    p1 = di * jnp.dot(x_ref[...], w1_ref[...],
                      preferred_element_type=jnp.float32)
    p1_ref[...] = p1
    p1h_ref[...] = p1.astype(jnp.bfloat16)
    dinv_ref[...] = di


def _tc_p1(x, w1, deg_col):
    grid = _N // _RB
    return pl.pallas_call(
        _p1_body,
        grid=(grid,),
        in_specs=[
            pl.BlockSpec((_RB, _D), lambda i: (i, 0)),
            pl.BlockSpec((_D, _D), lambda i: (0, 0)),
            pl.BlockSpec((_RB, 1), lambda i: (i, 0)),
        ],
        out_specs=[
            pl.BlockSpec((_RB, _D), lambda i: (i, 0)),
            pl.BlockSpec((_RB, _D), lambda i: (i, 0)),
            pl.BlockSpec((_RB, 1), lambda i: (i, 0)),
        ],
        out_shape=[
            jax.ShapeDtypeStruct((_N, _D), jnp.float32),
            jax.ShapeDtypeStruct((_N, _D), jnp.bfloat16),
            jax.ShapeDtypeStruct((_N, 1), jnp.float32),
        ],
    )(x, w1, deg_col)


def _p2_body(s1_ref, p1_ref, dinv_ref, b1_ref, w2_ref, p2_ref, p2h_ref):
    di = dinv_ref[...]
    s1 = (s1_ref[0].astype(jnp.float32) + s1_ref[1].astype(jnp.float32))
    h = di * (s1 + p1_ref[...]) + b1_ref[...]
    h = jnp.maximum(h, 0.0)
    p2 = di * jnp.dot(h, w2_ref[...], preferred_element_type=jnp.float32)
    p2_ref[...] = p2
    p2h_ref[...] = p2.astype(jnp.bfloat16)


def _tc_p2(s1, p1, dinv, b1_row, w2p):
    grid = _N // _RB
    return pl.pallas_call(
        _p2_body,
        grid=(grid,),
        in_specs=[
            pl.BlockSpec((2, _RB, _D), lambda i: (0, i, 0)),
            pl.BlockSpec((_RB, _D), lambda i: (i, 0)),
            pl.BlockSpec((_RB, 1), lambda i: (i, 0)),
            pl.BlockSpec((1, _D), lambda i: (0, 0)),
            pl.BlockSpec((_D, _CP), lambda i: (0, 0)),
        ],
        out_specs=[
            pl.BlockSpec((_RB, _CP), lambda i: (i, 0)),
            pl.BlockSpec((_RB, _CP), lambda i: (i, 0)),
        ],
        out_shape=[
            jax.ShapeDtypeStruct((_N, _CP), jnp.float32),
            jax.ShapeDtypeStruct((_N, _CP), jnp.bfloat16),
        ],
    )(s1, p1, dinv, b1_row, w2p)


def _final_body(s2_ref, p2_ref, dinv_ref, b2_ref, logp_ref, logits_ref):
    di = dinv_ref[...]
    s2 = (s2_ref[0].astype(jnp.float32) + s2_ref[1].astype(jnp.float32))
    lg = di * (s2 + p2_ref[...]) + b2_ref[...]
    icol = lax.broadcasted_iota(jnp.int32, (_RB, _CP), 1)
    neg = jnp.float32(-jnp.inf)
    lm = jnp.where(icol < _C, lg, neg)
    m = jnp.max(lm, axis=1, keepdims=True)
    e = jnp.where(icol < _C, jnp.exp(lm - m), 0.0)
    lse = m + jnp.log(jnp.sum(e, axis=1, keepdims=True))
    logp_ref[...] = (lg - lse)[:, :_C]
    logits_ref[...] = lg[:, :_C]


def _tc_final(s2, p2, dinv, b2_row):
    grid = _N // _RB
    return pl.pallas_call(
        _final_body,
        grid=(grid,),
        in_specs=[
            pl.BlockSpec((2, _RB, _CP), lambda i: (0, i, 0)),
            pl.BlockSpec((_RB, _CP), lambda i: (i, 0)),
            pl.BlockSpec((_RB, 1), lambda i: (i, 0)),
            pl.BlockSpec((1, _CP), lambda i: (0, 0)),
        ],
        out_specs=[
            pl.BlockSpec((_RB, _C), lambda i: (i, 0)),
            pl.BlockSpec((_RB, _C), lambda i: (i, 0)),
        ],
        out_shape=[
            jax.ShapeDtypeStruct((_N, _C), jnp.float32),
            jax.ShapeDtypeStruct((_N, _C), jnp.float32),
        ],
    )(s2, p2, dinv, b2_row)


def kernel(x, edge_index, W1, b1, W2, b2):
    edges = edge_index.astype(jnp.int32).reshape(2, _NCH, _K)

    degp = _degree(edges)
    deg_col = (degp[:_N] + degp[_NPD:_NPD + _N] + 1.0).reshape(_N, 1)

    p1, p1h, dinv = _tc_p1(x, W1, deg_col)

    s1 = _edge_scatter_d(p1h, edges)

    b1_row = b1.reshape(1, _D)
    w2p = jnp.pad(W2, ((0, 0), (0, _CP - _C)))
    p2, p2h = _tc_p2(s1, p1, dinv, b1_row, w2p)

    s2 = _edge_scatter_c(p2h, edges)

    b2_row = jnp.pad(b2, (0, _CP - _C)).reshape(1, _CP)
    logp, logits = _tc_final(s2, p2, dinv, b2_row)

    return (logp, logits)


# speedup vs baseline: 45.1613x; 1.0074x over previous
import functools

import jax
import jax.numpy as jnp
from jax import lax
from jax.experimental import pallas as pl
from jax.experimental.pallas import tpu as pltpu
from jax.experimental.pallas import tpu_sc as plsc

_N = 10000
_E = 320000
_D = 128
_C = 40
_CP = 64
_NPD = 10240
_NC = 2
_NS = 16
_NW = _NC * _NS
_K = 128
_NCH = _E // _K
_CPW = _NCH // _NW
_XTRA = _NCH - _CPW * _NW
_STRIP = _N // _NS
_RB = 2000

_MESH = dict(core_axis_name="c", subcore_axis_name="s",
             num_cores=_NC, num_subcores=_NS)


def _zero_fill_2d(ref, nrows, dim):
    zeros32 = jnp.zeros((32,), jnp.bfloat16)
    per_row = dim // 32

    def body(i, carry):
        ref[i // per_row, pl.ds((i % per_row) * 32, 32)] = zeros32
        return carry

    lax.fori_loop(0, nrows * per_row, body, 0)


def _zero_fill_1d(ref, n):
    zeros16 = jnp.zeros((16,), jnp.float32)

    def body(i, carry):
        ref[pl.ds(i * 16, 16)] = zeros16
        return carry

    lax.fori_loop(0, n // 16, body, 0)


def _zero_acc_strip(zsrc, acc, strip, dim):
    nz = _STRIP // _K
    for z in range(nz):
        pltpu.sync_copy(zsrc, acc.at[pl.ds(strip + z * _K, _K), :])
    rem = _STRIP % _K
    if rem:
        pltpu.sync_copy(zsrc.at[pl.ds(0, rem), :],
                        acc.at[pl.ds(strip + nz * _K, rem), :])


def _write_out_strip(acc, out_hbm, c, strip, stage, dim):
    nz = _STRIP // _K
    for z in range(nz):
        r0 = strip + z * _K
        pltpu.sync_copy(acc.at[pl.ds(r0, _K), :], stage)
        pltpu.sync_copy(stage, out_hbm.at[c, pl.ds(r0, _K), :])
    rem = _STRIP % _K
    if rem:
        r0 = strip + nz * _K
        pltpu.sync_copy(acc.at[pl.ds(r0, rem), :],
                        stage.at[pl.ds(0, rem), :])
        pltpu.sync_copy(stage.at[pl.ds(0, rem), :],
                        out_hbm.at[c, pl.ds(r0, rem), :])


def _make_edge_scatter_full(dim, nslot):
    assert _CPW % nslot == 0
    mesh = plsc.VectorSubcoreMesh(**_MESH)

    @functools.partial(
        pl.kernel,
        out_type=jax.ShapeDtypeStruct((_NC, _N, dim), jnp.bfloat16),
        mesh=mesh,
        scratch_types=[
            pltpu.VMEM((_CPW + 1, _K), jnp.int32),
            pltpu.VMEM((_CPW + 1, _K), jnp.int32),
            [pltpu.VMEM((_K, dim), jnp.bfloat16) for _ in range(nslot)],
            pltpu.VMEM_SHARED((_N, dim), jnp.bfloat16),
            [pltpu.SemaphoreType.DMA for _ in range(nslot)],
            [pltpu.SemaphoreType.DMA for _ in range(nslot)],
        ],
        compiler_params=pltpu.CompilerParams(use_tc_tiling_on_sc=False),
    )
    def scatter_kernel(p_hbm, edge_hbm, out_hbm,
                       row_v, col_v, bufs, acc, gsems, ssems):
        c = lax.axis_index("c")
        s = lax.axis_index("s")
        wid = c * _NS + s
        strip = s * _STRIP
        cbase = wid * _CPW

        pltpu.sync_copy(edge_hbm.at[0, pl.ds(cbase, _CPW), :],
                        row_v.at[pl.ds(0, _CPW), :])
        pltpu.sync_copy(edge_hbm.at[1, pl.ds(cbase, _CPW), :],
                        col_v.at[pl.ds(0, _CPW), :])
        _zero_fill_2d(bufs[0], _K, dim)
        _zero_acc_strip(bufs[0], acc, strip, dim)
        plsc.subcore_barrier()

        for b in range(nslot):
            pltpu.async_copy(p_hbm.at[row_v.at[b]], bufs[b], gsems[b])

        def body(r, carry):
            handles = []
            for b in range(nslot):
                j = r * nslot + b
                pltpu.make_async_copy(p_hbm.at[row_v.at[j]], bufs[b],
                                      gsems[b]).wait()
                handles.append(pltpu.async_copy(
                    bufs[b], acc.at[col_v.at[j]], ssems[b], add=True))
            for b in range(nslot):
                handles[b].wait()
                j2 = r * nslot + b + nslot

                @pl.when(j2 < _CPW)
                def _():
                    pltpu.async_copy(p_hbm.at[row_v.at[j2]], bufs[b],
                                     gsems[b])
            return carry

        lax.fori_loop(0, _CPW // nslot, body, 0)

        @pl.when(wid < _XTRA)
        def _():
            cid = _NW * _CPW + wid
            pltpu.sync_copy(edge_hbm.at[0, cid, :], row_v.at[_CPW])
            pltpu.sync_copy(edge_hbm.at[1, cid, :], col_v.at[_CPW])
            pltpu.async_copy(p_hbm.at[row_v.at[_CPW]], bufs[0],
                             gsems[0]).wait()
            pltpu.sync_copy(bufs[0], acc.at[col_v.at[_CPW]], add=True)

        plsc.subcore_barrier()
        _write_out_strip(acc, out_hbm, c, strip, bufs[0], dim)

    return scatter_kernel


def _make_degree():
    mesh = plsc.VectorSubcoreMesh(**_MESH)
    fire = 6

    @functools.partial(
        pl.kernel,
        out_type=jax.ShapeDtypeStruct((_NC * _NPD,), jnp.float32),
        mesh=mesh,
        scratch_types=[
            pltpu.VMEM((_CPW + 1, _K), jnp.int32),
            pltpu.VMEM((_K,), jnp.float32),
            pltpu.VMEM((_NPD // _NS,), jnp.float32),
            pltpu.VMEM_SHARED((_NPD,), jnp.float32),
            pltpu.SemaphoreType.DMA,
        ],
        compiler_params=pltpu.CompilerParams(use_tc_tiling_on_sc=False),
    )
    def degree_kernel(edge_hbm, out_hbm, col_v, ones_v, stage_v, acc, sem):
        c = lax.axis_index("c")
        s = lax.axis_index("s")
        wid = c * _NS + s
        dstrip = _NPD // _NS
        strip = s * dstrip

        ones16 = jnp.ones((16,), jnp.float32)
        for i in range(_K // 16):
            ones_v[pl.ds(i * 16, 16)] = ones16
        pltpu.sync_copy(edge_hbm.at[1, pl.ds(wid * _CPW, _CPW), :],
                        col_v.at[pl.ds(0, _CPW), :])
        _zero_fill_1d(stage_v, dstrip)
        pltpu.sync_copy(stage_v, acc.at[pl.ds(strip, dstrip)])
        plsc.subcore_barrier()

        def body(r, carry):
            handles = [
                pltpu.async_copy(ones_v, acc.at[col_v.at[r * fire + b]],
                                 sem, add=True)
                for b in range(fire)
            ]
            for h in handles:
                h.wait()
            return carry

        lax.fori_loop(0, _CPW // fire, body, 0)

        @pl.when(wid < _XTRA)
        def _():
            cid = _NW * _CPW + wid
            pltpu.sync_copy(edge_hbm.at[1, cid, :], col_v.at[_CPW])
            pltpu.sync_copy(ones_v, acc.at[col_v.at[_CPW]], add=True)

        plsc.subcore_barrier()
        pltpu.sync_copy(acc.at[pl.ds(strip, dstrip)], stage_v)
        pltpu.sync_copy(stage_v, out_hbm.at[pl.ds(c * _NPD + strip, dstrip)])

    return degree_kernel


_edge_scatter_d = _make_edge_scatter_full(_D, 6)
_edge_scatter_c = _make_edge_scatter_full(_CP, 13)
_degree = _make_degree()


def _p1_body(x_ref, w1_ref, deg_ref, p1_ref, p1h_ref, dinv_ref):
    di = lax.rsqrt(deg_ref[...])
    p1 = di * jnp.dot(x_ref[...], w1_ref[...],
                      preferred_element_type=jnp.float32)
    p1_ref[...] = p1
    p1h_ref[...] = p1.astype(jnp.bfloat16)
    dinv_ref[...] = di


def _tc_p1(x, w1, deg_col):
    grid = _N // _RB
    return pl.pallas_call(
        _p1_body,
        grid=(grid,),
        in_specs=[
            pl.BlockSpec((_RB, _D), lambda i: (i, 0)),
            pl.BlockSpec((_D, _D), lambda i: (0, 0)),
            pl.BlockSpec((_RB, 1), lambda i: (i, 0)),
        ],
        out_specs=[
            pl.BlockSpec((_RB, _D), lambda i: (i, 0)),
            pl.BlockSpec((_RB, _D), lambda i: (i, 0)),
            pl.BlockSpec((_RB, 1), lambda i: (i, 0)),
        ],
        out_shape=[
            jax.ShapeDtypeStruct((_N, _D), jnp.float32),
            jax.ShapeDtypeStruct((_N, _D), jnp.bfloat16),
            jax.ShapeDtypeStruct((_N, 1), jnp.float32),
        ],
    )(x, w1, deg_col)


def _p2_body(s1_ref, p1_ref, dinv_ref, b1_ref, w2_ref, p2_ref, p2h_ref):
    di = dinv_ref[...]
    s1 = (s1_ref[0].astype(jnp.float32) + s1_ref[1].astype(jnp.float32))
    h = di * (s1 + p1_ref[...]) + b1_ref[...]
    h = jnp.maximum(h, 0.0)
    p2 = di * jnp.dot(h, w2_ref[...], preferred_element_type=jnp.float32)
    p2_ref[...] = p2
    p2h_ref[...] = p2.astype(jnp.bfloat16)


def _tc_p2(s1, p1, dinv, b1_row, w2p):
    grid = _N // _RB
    return pl.pallas_call(
        _p2_body,
        grid=(grid,),
        in_specs=[
            pl.BlockSpec((2, _RB, _D), lambda i: (0, i, 0)),
            pl.BlockSpec((_RB, _D), lambda i: (i, 0)),
            pl.BlockSpec((_RB, 1), lambda i: (i, 0)),
            pl.BlockSpec((1, _D), lambda i: (0, 0)),
            pl.BlockSpec((_D, _CP), lambda i: (0, 0)),
        ],
        out_specs=[
            pl.BlockSpec((_RB, _CP), lambda i: (i, 0)),
            pl.BlockSpec((_RB, _CP), lambda i: (i, 0)),
        ],
        out_shape=[
            jax.ShapeDtypeStruct((_N, _CP), jnp.float32),
            jax.ShapeDtypeStruct((_N, _CP), jnp.bfloat16),
        ],
    )(s1, p1, dinv, b1_row, w2p)


def _final_body(s2_ref, p2_ref, dinv_ref, b2_ref, logp_ref, logits_ref):
    di = dinv_ref[...]
    s2 = (s2_ref[0].astype(jnp.float32) + s2_ref[1].astype(jnp.float32))
    lg = di * (s2 + p2_ref[...]) + b2_ref[...]
    icol = lax.broadcasted_iota(jnp.int32, (_RB, _CP), 1)
    neg = jnp.float32(-jnp.inf)
    lm = jnp.where(icol < _C, lg, neg)
    m = jnp.max(lm, axis=1, keepdims=True)
    e = jnp.where(icol < _C, jnp.exp(lm - m), 0.0)
    lse = m + jnp.log(jnp.sum(e, axis=1, keepdims=True))
    logp_ref[...] = (lg - lse)[:, :_C]
    logits_ref[...] = lg[:, :_C]


def _tc_final(s2, p2, dinv, b2_row):
    grid = _N // _RB
    return pl.pallas_call(
        _final_body,
        grid=(grid,),
        in_specs=[
            pl.BlockSpec((2, _RB, _CP), lambda i: (0, i, 0)),
            pl.BlockSpec((_RB, _CP), lambda i: (i, 0)),
            pl.BlockSpec((_RB, 1), lambda i: (i, 0)),
            pl.BlockSpec((1, _CP), lambda i: (0, 0)),
        ],
        out_specs=[
            pl.BlockSpec((_RB, _C), lambda i: (i, 0)),
            pl.BlockSpec((_RB, _C), lambda i: (i, 0)),
        ],
        out_shape=[
            jax.ShapeDtypeStruct((_N, _C), jnp.float32),
            jax.ShapeDtypeStruct((_N, _C), jnp.float32),
        ],
    )(s2, p2, dinv, b2_row)


def kernel(x, edge_index, W1, b1, W2, b2):
    edges = edge_index.astype(jnp.int32).reshape(2, _NCH, _K)

    degp = _degree(edges)
    deg_col = (degp[:_N] + degp[_NPD:_NPD + _N] + 1.0).reshape(_N, 1)

    p1, p1h, dinv = _tc_p1(x, W1, deg_col)

    s1 = _edge_scatter_d(p1h, edges)

    b1_row = b1.reshape(1, _D)
    w2p = jnp.pad(W2, ((0, 0), (0, _CP - _C)))
    p2, p2h = _tc_p2(s1, p1, dinv, b1_row, w2p)

    s2 = _edge_scatter_c(p2h, edges)

    b2_row = jnp.pad(b2, (0, _CP - _C)).reshape(1, _CP)
    logp, logits = _tc_final(s2, p2, dinv, b2_row)

    return (logp, logits)
